# baseline (speedup 1.0000x reference)
import functools

import jax
import jax.numpy as jnp
from jax import lax
from jax.experimental import pallas as pl
from jax.experimental.pallas import tpu as pltpu
from jax.experimental.pallas import tpu_sc as plsc

NUM_USERS = 100000
EMBED_DIM = 64
BATCH = 4096

_info = plsc.get_sparse_core_info()
_NC, _NS, _L = _info.num_cores, _info.num_subcores, _info.num_lanes
_NW = _NC * _NS
_NCOLS = (NUM_USERS + 127) // 128
_LASTW = NUM_USERS - (_NCOLS - 1) * 128
_COLS_LO = _NCOLS // _NW
_COLS_EXTRA = _NCOLS - _COLS_LO * _NW
_NRB = 16


def _make_gather():
    mesh = plsc.VectorSubcoreMesh(core_axis_name="c", subcore_axis_name="s")

    @functools.partial(
        pl.kernel,
        mesh=mesh,
        out_type=jax.ShapeDtypeStruct((BATCH * EMBED_DIM,), jnp.float32),
        scratch_types=[
            pltpu.VMEM((BATCH,), jnp.int32),
            pltpu.VMEM((BATCH,), jnp.int32),
            pltpu.VMEM((BATCH,), jnp.int32),
            pltpu.VMEM((BATCH,), jnp.int32),
            pltpu.VMEM((EMBED_DIM, 128), jnp.float32),
            pltpu.VMEM((_NRB * EMBED_DIM,), jnp.float32),
            pltpu.SemaphoreType.DMA,
        ],
        compiler_params=pltpu.CompilerParams(needs_layout_passes=False),
    )
    def gather_kernel(idx_hbm, tab_t_hbm, tail_hbm, out_hbm, idx_v, my_b,
                      my_c, my_l, tblk, rowring, sem_out):
        wid = lax.axis_index("s") * _NC + lax.axis_index("c")
        nc = jnp.where(wid < _COLS_EXTRA, _COLS_LO + 1, _COLS_LO)
        lo = wid * _COLS_LO + jnp.minimum(wid, _COLS_EXTRA)

        pltpu.sync_copy(idx_hbm, idx_v)

        lane = lax.iota(jnp.int32, _L)

        def scan_body(v, off):
            vec = idx_v[pl.ds(v * _L, _L)]
            c = jax.lax.shift_right_logical(vec, 7)
            low = jnp.bitwise_and(vec, 127)
            b = lane + v * _L
            m = jnp.logical_and(c >= lo, c < lo + nc)
            m01 = jnp.where(m, jnp.full((_L,), 1, jnp.int32),
                            jnp.full((_L,), 0, jnp.int32))
            incl = plsc.cumsum(m01)
            pos = jnp.full((_L,), off, jnp.int32) + incl - m01
            plsc.store_scatter(my_b, [pos], b, mask=m)
            plsc.store_scatter(my_c, [pos], c, mask=m)
            plsc.store_scatter(my_l, [pos], low, mask=m)
            return off + jnp.sum(m01)

        m_total = lax.fori_loop(0, BATCH // _L, scan_body, jnp.int32(0),
                                unroll=False)
        nvec = (m_total + _L - 1) // _L

        def col_body(i, u_tot):
            cblk = lo + i
            coff = pl.multiple_of(cblk * 128, 128)

            @pl.when(cblk == _NCOLS - 1)
            def _():
                pltpu.sync_copy(tail_hbm, tblk)

            @pl.when(cblk != _NCOLS - 1)
            def _():
                pltpu.sync_copy(tab_t_hbm.at[:, pl.ds(coff, 128)], tblk)

            def match_body(v, u_tot):
                cvec = my_c[pl.ds(v * _L, _L)]
                valid = (lane + v * _L) < m_total
                m0 = jnp.logical_and(cvec == cblk, valid)

                one_v = jnp.full((_L,), 1, jnp.int32)
                zero_v = jnp.full((_L,), 0, jnp.int32)
                sent_v = jnp.full((_L,), _L, jnp.int32)

                def have_more(state):
                    m, _ = state
                    return jnp.sum(jnp.where(m, one_v, zero_v)) > 0

                def emit_one(state):
                    m, u = state
                    f_s = jnp.min(jnp.where(m, lane, sent_v))
                    f = jnp.full((_L,), f_s, jnp.int32)
                    p = f + v * _L
                    lspl = plsc.load_gather(my_l, [p])
                    bspl = plsc.load_gather(my_b, [p])
                    slot = jax.lax.rem(u, jnp.int32(_NRB))

                    @pl.when(u >= _NRB)
                    def _():
                        pltpu.make_async_copy(
                            out_hbm.at[pl.ds(0, EMBED_DIM)],
                            rowring.at[pl.ds(0, EMBED_DIM)],
                            sem_out,
                        ).wait()

                    roff = pl.multiple_of(slot * EMBED_DIM, 8)
                    for g in range(EMBED_DIM // _L):
                        dvec = lane + g * _L
                        col = plsc.load_gather(tblk, [dvec, lspl])
                        rowring[pl.ds(roff + g * _L, _L)] = col
                    b0 = bspl[0]
                    boff = pl.multiple_of(b0 * EMBED_DIM, 8)
                    pltpu.async_copy(
                        rowring.at[pl.ds(roff, EMBED_DIM)],
                        out_hbm.at[pl.ds(boff, EMBED_DIM)],
                        sem_out,
                    )
                    m = jnp.logical_and(m, lane != f)
                    return m, u + 1

                _, u_tot = lax.while_loop(have_more, emit_one, (m0, u_tot))
                return u_tot

            return lax.fori_loop(0, nvec, match_body, u_tot, unroll=False)

        u_tot = lax.fori_loop(0, nc, col_body, jnp.int32(0), unroll=False)

        def drain_body(_, __):
            pltpu.make_async_copy(
                out_hbm.at[pl.ds(0, EMBED_DIM)],
                rowring.at[pl.ds(0, EMBED_DIM)],
                sem_out,
            ).wait()
            return ()

        lax.fori_loop(0, jnp.minimum(u_tot, _NRB), drain_body, (),
                      unroll=False)

    return gather_kernel


_gather = _make_gather()


def kernel(user_idx, table):
    tail_pad = jnp.pad(table[NUM_USERS - _LASTW:].T,
                       ((0, 0), (0, 128 - _LASTW)))
    out1d = _gather(user_idx.astype(jnp.int32), table.T, tail_pad)
    return out1d.reshape(BATCH, EMBED_DIM)


# speedup vs baseline: 1.4744x; 1.4744x over previous
import functools

import jax
import jax.numpy as jnp
from jax import lax
from jax.experimental import pallas as pl
from jax.experimental.pallas import tpu as pltpu
from jax.experimental.pallas import tpu_sc as plsc

NUM_USERS = 100000
EMBED_DIM = 64
BATCH = 4096

_info = plsc.get_sparse_core_info()
_NC, _NS, _L = _info.num_cores, _info.num_subcores, _info.num_lanes
_NW = _NC * _NS
_NCOLS = (NUM_USERS + 127) // 128
_LASTW = NUM_USERS - (_NCOLS - 1) * 128
_COLS_LO = _NCOLS // _NW
_COLS_EXTRA = _NCOLS - _COLS_LO * _NW
_NRB = 16
_NBUF = 4


def _make_gather():
    mesh = plsc.VectorSubcoreMesh(core_axis_name="c", subcore_axis_name="s")

    @functools.partial(
        pl.kernel,
        mesh=mesh,
        out_type=jax.ShapeDtypeStruct((BATCH * EMBED_DIM,), jnp.float32),
        scratch_types=[
            pltpu.VMEM((BATCH,), jnp.int32),
            pltpu.VMEM((BATCH,), jnp.int32),
            pltpu.VMEM((BATCH,), jnp.int32),
            pltpu.VMEM((BATCH,), jnp.int32),
            pltpu.VMEM((_NBUF, EMBED_DIM, 128), jnp.float32),
            pltpu.VMEM((_NRB * EMBED_DIM,), jnp.float32),
            pltpu.SemaphoreType.DMA,
            pltpu.SemaphoreType.DMA,
        ],
        compiler_params=pltpu.CompilerParams(needs_layout_passes=False),
    )
    def gather_kernel(idx_hbm, tab_t_hbm, tail_hbm, out_hbm, idx_v, my_b,
                      my_c, my_l, tblk, rowring, sem_in, sem_out):
        wid = lax.axis_index("s") * _NC + lax.axis_index("c")
        nc = jnp.where(wid < _COLS_EXTRA, _COLS_LO + 1, _COLS_LO)
        lo = wid * _COLS_LO + jnp.minimum(wid, _COLS_EXTRA)

        pltpu.sync_copy(idx_hbm, idx_v)

        lane = lax.iota(jnp.int32, _L)

        def scan_body(v, off):
            vec = idx_v[pl.ds(v * _L, _L)]
            c = jax.lax.shift_right_logical(vec, 7)
            low = jnp.bitwise_and(vec, 127)
            b = lane + v * _L
            m = jnp.logical_and(c >= lo, c < lo + nc)
            m01 = jnp.where(m, jnp.full((_L,), 1, jnp.int32),
                            jnp.full((_L,), 0, jnp.int32))
            incl = plsc.cumsum(m01)
            pos = jnp.full((_L,), off, jnp.int32) + incl - m01
            plsc.store_scatter(my_b, [pos], b, mask=m)
            plsc.store_scatter(my_c, [pos], c, mask=m)
            plsc.store_scatter(my_l, [pos], low, mask=m)
            return off + jnp.sum(m01)

        m_total = lax.fori_loop(0, BATCH // _L, scan_body, jnp.int32(0),
                                unroll=False)
        nvec = (m_total + _L - 1) // _L

        def start_blk(i):
            par = jax.lax.rem(i, jnp.int32(_NBUF))
            cblk = lo + i
            coff = pl.multiple_of(cblk * 128, 128)

            @pl.when(cblk == _NCOLS - 1)
            def _():
                pltpu.async_copy(tail_hbm, tblk.at[par], sem_in)

            @pl.when(cblk != _NCOLS - 1)
            def _():
                pltpu.async_copy(tab_t_hbm.at[:, pl.ds(coff, 128)],
                                 tblk.at[par], sem_in)

        def prime_body(i, _):
            @pl.when(i < nc)
            def _():
                start_blk(i)
            return ()

        lax.fori_loop(0, _NBUF - 1, prime_body, (), unroll=True)

        def col_body(i, u_tot):
            par = jax.lax.rem(i, jnp.int32(_NBUF))

            @pl.when(i + _NBUF - 1 < nc)
            def _():
                start_blk(i + _NBUF - 1)

            pltpu.make_async_copy(
                tab_t_hbm.at[:, pl.ds(0, 128)], tblk.at[par], sem_in
            ).wait()
            cblk = lo + i
            par_v = jnp.full((_L,), par, jnp.int32)

            def match_body(v, u_tot):
                cvec = my_c[pl.ds(v * _L, _L)]
                valid = (lane + v * _L) < m_total
                m0 = jnp.logical_and(cvec == cblk, valid)

                one_v = jnp.full((_L,), 1, jnp.int32)
                zero_v = jnp.full((_L,), 0, jnp.int32)
                sent_v = jnp.full((_L,), _L, jnp.int32)

                def have_more(state):
                    m, _ = state
                    return jnp.sum(jnp.where(m, one_v, zero_v)) > 0

                def emit_one(state):
                    m, u = state
                    f_s = jnp.min(jnp.where(m, lane, sent_v))
                    f = jnp.full((_L,), f_s, jnp.int32)
                    p = f + v * _L
                    lspl = plsc.load_gather(my_l, [p])
                    bspl = plsc.load_gather(my_b, [p])
                    slot = jax.lax.rem(u, jnp.int32(_NRB))

                    @pl.when(u >= _NRB)
                    def _():
                        pltpu.make_async_copy(
                            out_hbm.at[pl.ds(0, EMBED_DIM)],
                            rowring.at[pl.ds(0, EMBED_DIM)],
                            sem_out,
                        ).wait()

                    roff = pl.multiple_of(slot * EMBED_DIM, 8)
                    for g in range(EMBED_DIM // _L):
                        dvec = lane + g * _L
                        col = plsc.load_gather(tblk, [par_v, dvec, lspl])
                        rowring[pl.ds(roff + g * _L, _L)] = col
                    b0 = bspl[0]
                    boff = pl.multiple_of(b0 * EMBED_DIM, 8)
                    pltpu.async_copy(
                        rowring.at[pl.ds(roff, EMBED_DIM)],
                        out_hbm.at[pl.ds(boff, EMBED_DIM)],
                        sem_out,
                    )
                    m = jnp.logical_and(m, lane != f)
                    return m, u + 1

                _, u_tot = lax.while_loop(have_more, emit_one, (m0, u_tot))
                return u_tot

            return lax.fori_loop(0, nvec, match_body, u_tot, unroll=False)

        u_tot = lax.fori_loop(0, nc, col_body, jnp.int32(0), unroll=False)

        def drain_body(_, __):
            pltpu.make_async_copy(
                out_hbm.at[pl.ds(0, EMBED_DIM)],
                rowring.at[pl.ds(0, EMBED_DIM)],
                sem_out,
            ).wait()
            return ()

        lax.fori_loop(0, jnp.minimum(u_tot, _NRB), drain_body, (),
                      unroll=False)

    return gather_kernel


_gather = _make_gather()


def kernel(user_idx, table):
    tail_pad = jnp.pad(table[NUM_USERS - _LASTW:].T,
                       ((0, 0), (0, 128 - _LASTW)))
    out1d = _gather(user_idx.astype(jnp.int32), table.T, tail_pad)
    return out1d.reshape(BATCH, EMBED_DIM)


# speedup vs baseline: 1.5795x; 1.0713x over previous
import functools

import jax
import jax.numpy as jnp
from jax import lax
from jax.experimental import pallas as pl
from jax.experimental.pallas import tpu as pltpu
from jax.experimental.pallas import tpu_sc as plsc

NUM_USERS = 100000
EMBED_DIM = 64
BATCH = 4096

_info = plsc.get_sparse_core_info()
_NC, _NS, _L = _info.num_cores, _info.num_subcores, _info.num_lanes
_NW = _NC * _NS
_CW = 256
_NCH = (NUM_USERS + _CW - 1) // _CW
_LASTW = NUM_USERS - (_NCH - 1) * _CW - 128
_CH_LO = _NCH // _NW
_CH_EXTRA = _NCH - _CH_LO * _NW
_NRB = 16
_NBUF = 4


def _make_gather():
    mesh = plsc.VectorSubcoreMesh(core_axis_name="c", subcore_axis_name="s")

    @functools.partial(
        pl.kernel,
        mesh=mesh,
        out_type=jax.ShapeDtypeStruct((BATCH * EMBED_DIM,), jnp.float32),
        scratch_types=[
            pltpu.VMEM((BATCH,), jnp.int32),
            pltpu.VMEM((BATCH,), jnp.int32),
            pltpu.VMEM((BATCH,), jnp.int32),
            pltpu.VMEM((BATCH,), jnp.int32),
            pltpu.VMEM((_NBUF, EMBED_DIM, _CW), jnp.float32),
            pltpu.VMEM((_NRB * EMBED_DIM,), jnp.float32),
            pltpu.SemaphoreType.DMA,
            pltpu.SemaphoreType.DMA,
        ],
        compiler_params=pltpu.CompilerParams(needs_layout_passes=False),
    )
    def gather_kernel(idx_hbm, tab_t_hbm, tail_hbm, out_hbm, idx_v, my_b,
                      my_c, my_l, tblk, rowring, sem_in, sem_out):
        wid = lax.axis_index("s") * _NC + lax.axis_index("c")
        nch = jnp.where(wid < _CH_EXTRA, _CH_LO + 1, _CH_LO)
        lo = wid * _CH_LO + jnp.minimum(wid, _CH_EXTRA)

        def start_blk(i):
            par = jax.lax.rem(i, jnp.int32(_NBUF))
            cblk = lo + i
            coff = pl.multiple_of(cblk * _CW, 128)

            @pl.when(cblk == _NCH - 1)
            def _():
                pltpu.async_copy(tab_t_hbm.at[:, pl.ds(coff, 128)],
                                 tblk.at[par, :, pl.ds(0, 128)], sem_in)
                pltpu.async_copy(tail_hbm,
                                 tblk.at[par, :, pl.ds(128, 128)], sem_in)

            @pl.when(cblk != _NCH - 1)
            def _():
                pltpu.async_copy(tab_t_hbm.at[:, pl.ds(coff, _CW)],
                                 tblk.at[par], sem_in)

        def prime_body(i, _):
            @pl.when(i < nch)
            def _():
                start_blk(i)
            return ()

        lax.fori_loop(0, _NBUF - 1, prime_body, (), unroll=True)

        pltpu.sync_copy(idx_hbm, idx_v)

        lane = lax.iota(jnp.int32, _L)
        one_v = jnp.full((_L,), 1, jnp.int32)
        zero_v = jnp.full((_L,), 0, jnp.int32)
        sent_v = jnp.full((_L,), _L, jnp.int32)

        def scan_body(v, off):
            vec = idx_v[pl.ds(v * _L, _L)]
            c = jax.lax.shift_right_logical(vec, 8)
            low = jnp.bitwise_and(vec, _CW - 1)
            b = lane + v * _L
            m = jnp.logical_and(c >= lo, c < lo + nch)
            m01 = jnp.where(m, one_v, zero_v)
            incl = plsc.cumsum(m01)
            pos = jnp.full((_L,), off, jnp.int32) + incl - m01
            plsc.store_scatter(my_b, [pos], b, mask=m)
            plsc.store_scatter(my_c, [pos], c, mask=m)
            plsc.store_scatter(my_l, [pos], low, mask=m)
            return off + jnp.sum(m01)

        m_total = lax.fori_loop(0, BATCH // _L, scan_body, jnp.int32(0),
                                unroll=False)
        nvec = (m_total + _L - 1) // _L

        def col_body(i, u_tot):
            par = jax.lax.rem(i, jnp.int32(_NBUF))

            @pl.when(i + _NBUF - 1 < nch)
            def _():
                start_blk(i + _NBUF - 1)

            pltpu.make_async_copy(
                tab_t_hbm.at[:, pl.ds(0, _CW)], tblk.at[par], sem_in
            ).wait()
            cblk = lo + i
            par_v = jnp.full((_L,), par, jnp.int32)

            def match_body(v, u_tot):
                cvec = my_c[pl.ds(v * _L, _L)]
                valid = (lane + v * _L) < m_total
                m0 = jnp.logical_and(cvec == cblk, valid)

                def have_more(state):
                    m, _ = state
                    return jnp.sum(jnp.where(m, one_v, zero_v)) > 0

                def emit_one(state):
                    m, u = state
                    f_s = jnp.min(jnp.where(m, lane, sent_v))
                    f = jnp.full((_L,), f_s, jnp.int32)
                    p = f + v * _L
                    lspl = plsc.load_gather(my_l, [p])
                    bspl = plsc.load_gather(my_b, [p])
                    slot = jax.lax.rem(u, jnp.int32(_NRB))

                    @pl.when(u >= _NRB)
                    def _():
                        pltpu.make_async_copy(
                            out_hbm.at[pl.ds(0, EMBED_DIM)],
                            rowring.at[pl.ds(0, EMBED_DIM)],
                            sem_out,
                        ).wait()

                    roff = pl.multiple_of(slot * EMBED_DIM, 8)
                    for g in range(EMBED_DIM // _L):
                        dvec = lane + g * _L
                        col = plsc.load_gather(tblk, [par_v, dvec, lspl])
                        rowring[pl.ds(roff + g * _L, _L)] = col
                    b0 = bspl[0]
                    boff = pl.multiple_of(b0 * EMBED_DIM, 8)
                    pltpu.async_copy(
                        rowring.at[pl.ds(roff, EMBED_DIM)],
                        out_hbm.at[pl.ds(boff, EMBED_DIM)],
                        sem_out,
                    )
                    m = jnp.logical_and(m, lane != f)
                    return m, u + 1

                _, u_tot = lax.while_loop(have_more, emit_one, (m0, u_tot))
                return u_tot

            return lax.fori_loop(0, nvec, match_body, u_tot, unroll=False)

        u_tot = lax.fori_loop(0, nch, col_body, jnp.int32(0), unroll=False)

        def drain_body(_, __):
            pltpu.make_async_copy(
                out_hbm.at[pl.ds(0, EMBED_DIM)],
                rowring.at[pl.ds(0, EMBED_DIM)],
                sem_out,
            ).wait()
            return ()

        lax.fori_loop(0, jnp.minimum(u_tot, _NRB), drain_body, (),
                      unroll=False)

    return gather_kernel


_gather = _make_gather()


def kernel(user_idx, table):
    tail_pad = jnp.pad(table[(_NCH - 1) * _CW + 128:].T,
                       ((0, 0), (0, 128 - _LASTW)))
    out1d = _gather(user_idx.astype(jnp.int32), table.T, tail_pad)
    return out1d.reshape(BATCH, EMBED_DIM)


# speedup vs baseline: 1.6062x; 1.0169x over previous
import functools

import jax
import jax.numpy as jnp
from jax import lax
from jax.experimental import pallas as pl
from jax.experimental.pallas import tpu as pltpu
from jax.experimental.pallas import tpu_sc as plsc

NUM_USERS = 100000
EMBED_DIM = 64
BATCH = 4096

_info = plsc.get_sparse_core_info()
_NC, _NS, _L = _info.num_cores, _info.num_subcores, _info.num_lanes
_NW = _NC * _NS
_CW = 256
_NCH = (NUM_USERS + _CW - 1) // _CW
_LASTW = NUM_USERS - (_NCH - 1) * _CW - 128
_CH_LO = _NCH // _NW
_CH_EXTRA = _NCH - _CH_LO * _NW
_NRB = 16
_NBUF = 4


def _make_gather():
    mesh = plsc.VectorSubcoreMesh(core_axis_name="c", subcore_axis_name="s")

    @functools.partial(
        pl.kernel,
        mesh=mesh,
        out_type=jax.ShapeDtypeStruct((BATCH * EMBED_DIM,), jnp.float32),
        scratch_types=[
            pltpu.VMEM((BATCH,), jnp.int32),
            pltpu.VMEM((BATCH,), jnp.int32),
            pltpu.VMEM((BATCH,), jnp.int32),
            pltpu.VMEM((BATCH,), jnp.int32),
            pltpu.VMEM((_NBUF, EMBED_DIM, _CW), jnp.float32),
            pltpu.VMEM((_NRB * EMBED_DIM,), jnp.float32),
            pltpu.SemaphoreType.DMA,
            pltpu.SemaphoreType.DMA,
        ],
        compiler_params=pltpu.CompilerParams(needs_layout_passes=False),
    )
    def gather_kernel(idx_hbm, tab_t_hbm, tail_hbm, out_hbm, idx_v, my_b,
                      my_c, my_l, tblk, rowring, sem_in, sem_out):
        wid = lax.axis_index("s") * _NC + lax.axis_index("c")
        nch = jnp.where(wid < _CH_EXTRA, _CH_LO + 1, _CH_LO)
        lo = wid * _CH_LO + jnp.minimum(wid, _CH_EXTRA)

        def start_blk(i):
            par = jax.lax.rem(i, jnp.int32(_NBUF))
            cblk = lo + i
            coff = pl.multiple_of(cblk * _CW, 128)

            @pl.when(cblk == _NCH - 1)
            def _():
                pltpu.async_copy(tab_t_hbm.at[:, pl.ds(coff, 128)],
                                 tblk.at[par, :, pl.ds(0, 128)], sem_in)
                pltpu.async_copy(tail_hbm,
                                 tblk.at[par, :, pl.ds(128, 128)], sem_in)

            @pl.when(cblk != _NCH - 1)
            def _():
                pltpu.async_copy(tab_t_hbm.at[:, pl.ds(coff, _CW)],
                                 tblk.at[par], sem_in)

        def prime_body(i, _):
            @pl.when(i < nch)
            def _():
                start_blk(i)
            return ()

        lax.fori_loop(0, _NBUF - 1, prime_body, (), unroll=True)

        pltpu.sync_copy(idx_hbm, idx_v)

        lane = lax.iota(jnp.int32, _L)
        one_v = jnp.full((_L,), 1, jnp.int32)
        zero_v = jnp.full((_L,), 0, jnp.int32)
        sent_v = jnp.full((_L,), _L, jnp.int32)

        def scan_body(v, off):
            vec = idx_v[pl.ds(v * _L, _L)]
            c = jax.lax.shift_right_logical(vec, 8)
            low = jnp.bitwise_and(vec, _CW - 1)
            b = lane + v * _L
            m = jnp.logical_and(c >= lo, c < lo + nch)
            m01 = jnp.where(m, one_v, zero_v)
            incl = plsc.cumsum(m01)
            pos = jnp.full((_L,), off, jnp.int32) + incl - m01
            plsc.store_scatter(my_b, [pos], b, mask=m)
            plsc.store_scatter(my_c, [pos], c, mask=m)
            plsc.store_scatter(my_l, [pos], low, mask=m)
            return off + jnp.sum(m01)

        m_total = lax.fori_loop(0, BATCH // _L, scan_body, jnp.int32(0),
                                unroll=4)
        nvec = (m_total + _L - 1) // _L

        def col_body(i, u_tot):
            par = jax.lax.rem(i, jnp.int32(_NBUF))

            @pl.when(i + _NBUF - 1 < nch)
            def _():
                start_blk(i + _NBUF - 1)

            pltpu.make_async_copy(
                tab_t_hbm.at[:, pl.ds(0, _CW)], tblk.at[par], sem_in
            ).wait()
            cblk = lo + i
            par_v = jnp.full((_L,), par, jnp.int32)

            def match_body(v, u_tot):
                cvec = my_c[pl.ds(v * _L, _L)]
                valid = (lane + v * _L) < m_total
                m0 = jnp.logical_and(cvec == cblk, valid)
                s0 = jnp.sum(jnp.where(m0, one_v, zero_v))

                def emit_one(_, state):
                    m, u = state
                    f_s = jnp.min(jnp.where(m, lane, sent_v))
                    f = jnp.full((_L,), f_s, jnp.int32)
                    p = f + v * _L
                    lspl = plsc.load_gather(my_l, [p])
                    bspl = plsc.load_gather(my_b, [p])
                    slot = jax.lax.rem(u, jnp.int32(_NRB))

                    @pl.when(u >= _NRB)
                    def _():
                        pltpu.make_async_copy(
                            out_hbm.at[pl.ds(0, EMBED_DIM)],
                            rowring.at[pl.ds(0, EMBED_DIM)],
                            sem_out,
                        ).wait()

                    roff = pl.multiple_of(slot * EMBED_DIM, 8)
                    for g in range(EMBED_DIM // _L):
                        dvec = lane + g * _L
                        col = plsc.load_gather(tblk, [par_v, dvec, lspl])
                        rowring[pl.ds(roff + g * _L, _L)] = col
                    b0 = bspl[0]
                    boff = pl.multiple_of(b0 * EMBED_DIM, 8)
                    pltpu.async_copy(
                        rowring.at[pl.ds(roff, EMBED_DIM)],
                        out_hbm.at[pl.ds(boff, EMBED_DIM)],
                        sem_out,
                    )
                    m = jnp.logical_and(m, lane != f)
                    return m, u + 1

                _, u_tot = lax.fori_loop(0, s0, emit_one, (m0, u_tot),
                                         unroll=False)
                return u_tot

            return lax.fori_loop(0, nvec, match_body, u_tot, unroll=False)

        u_tot = lax.fori_loop(0, nch, col_body, jnp.int32(0), unroll=False)

        def drain_body(_, __):
            pltpu.make_async_copy(
                out_hbm.at[pl.ds(0, EMBED_DIM)],
                rowring.at[pl.ds(0, EMBED_DIM)],
                sem_out,
            ).wait()
            return ()

        lax.fori_loop(0, jnp.minimum(u_tot, _NRB), drain_body, (),
                      unroll=False)

    return gather_kernel


_gather = _make_gather()


def kernel(user_idx, table):
    tail_pad = jnp.pad(table[(_NCH - 1) * _CW + 128:].T,
                       ((0, 0), (0, 128 - _LASTW)))
    out1d = _gather(user_idx.astype(jnp.int32), table.T, tail_pad)
    return out1d.reshape(BATCH, EMBED_DIM)


# speedup vs baseline: 1.6545x; 1.0301x over previous
import functools

import jax
import jax.numpy as jnp
from jax import lax
from jax.experimental import pallas as pl
from jax.experimental.pallas import tpu as pltpu
from jax.experimental.pallas import tpu_sc as plsc

NUM_USERS = 100000
EMBED_DIM = 64
BATCH = 4096

_info = plsc.get_sparse_core_info()
_NC, _NS, _L = _info.num_cores, _info.num_subcores, _info.num_lanes
_NW = _NC * _NS
_CW = 512
_NCH = (NUM_USERS + _CW - 1) // _CW
_LASTW = NUM_USERS - (_NCH - 1) * _CW - 128
_CH_LO = _NCH // _NW
_CH_EXTRA = _NCH - _CH_LO * _NW
_NRB = 16
_NBUF = 3


def _make_gather():
    mesh = plsc.VectorSubcoreMesh(core_axis_name="c", subcore_axis_name="s")

    @functools.partial(
        pl.kernel,
        mesh=mesh,
        out_type=jax.ShapeDtypeStruct((BATCH * EMBED_DIM,), jnp.float32),
        scratch_types=[
            pltpu.VMEM((BATCH,), jnp.int32),
            pltpu.VMEM((BATCH,), jnp.int32),
            pltpu.VMEM((BATCH,), jnp.int32),
            pltpu.VMEM((BATCH,), jnp.int32),
            pltpu.VMEM((_NBUF, EMBED_DIM, _CW), jnp.float32),
            pltpu.VMEM((_NRB * EMBED_DIM,), jnp.float32),
            pltpu.SemaphoreType.DMA,
            pltpu.SemaphoreType.DMA,
        ],
        compiler_params=pltpu.CompilerParams(needs_layout_passes=False),
    )
    def gather_kernel(idx_hbm, tab_t_hbm, tail_hbm, out_hbm, idx_v, my_b,
                      my_c, my_l, tblk, rowring, sem_in, sem_out):
        wid = lax.axis_index("s") * _NC + lax.axis_index("c")
        nch = jnp.where(wid < _CH_EXTRA, _CH_LO + 1, _CH_LO)
        lo = wid * _CH_LO + jnp.minimum(wid, _CH_EXTRA)

        def start_blk(i):
            par = jax.lax.rem(i, jnp.int32(_NBUF))
            cblk = lo + i
            coff = pl.multiple_of(cblk * _CW, 128)

            @pl.when(cblk == _NCH - 1)
            def _():
                pltpu.async_copy(tab_t_hbm.at[:, pl.ds(coff, 128)],
                                 tblk.at[par, :, pl.ds(0, 128)], sem_in)
                pltpu.async_copy(tail_hbm,
                                 tblk.at[par, :, pl.ds(128, 128)], sem_in)

            @pl.when(cblk != _NCH - 1)
            def _():
                pltpu.async_copy(tab_t_hbm.at[:, pl.ds(coff, _CW)],
                                 tblk.at[par], sem_in)

        def prime_body(i, _):
            @pl.when(i < nch)
            def _():
                start_blk(i)
            return ()

        lax.fori_loop(0, _NBUF - 1, prime_body, (), unroll=True)

        pltpu.sync_copy(idx_hbm, idx_v)

        lane = lax.iota(jnp.int32, _L)
        one_v = jnp.full((_L,), 1, jnp.int32)
        zero_v = jnp.full((_L,), 0, jnp.int32)
        sent_v = jnp.full((_L,), _L, jnp.int32)

        def scan_body(v, off):
            vec = idx_v[pl.ds(v * _L, _L)]
            c = jax.lax.shift_right_logical(vec, 9)
            low = jnp.bitwise_and(vec, _CW - 1)
            b = lane + v * _L
            m = jnp.logical_and(c >= lo, c < lo + nch)
            m01 = jnp.where(m, one_v, zero_v)
            incl = plsc.cumsum(m01)
            pos = jnp.full((_L,), off, jnp.int32) + incl - m01
            plsc.store_scatter(my_b, [pos], b, mask=m)
            plsc.store_scatter(my_c, [pos], c, mask=m)
            plsc.store_scatter(my_l, [pos], low, mask=m)
            return off + jnp.sum(m01)

        m_total = lax.fori_loop(0, BATCH // _L, scan_body, jnp.int32(0),
                                unroll=4)
        nvec = (m_total + _L - 1) // _L

        def col_body(i, u_tot):
            par = jax.lax.rem(i, jnp.int32(_NBUF))

            @pl.when(i + _NBUF - 1 < nch)
            def _():
                start_blk(i + _NBUF - 1)

            @pl.when((lo + i) == _NCH - 1)
            def _():
                pltpu.make_async_copy(
                    tab_t_hbm.at[:, pl.ds(0, 256)],
                    tblk.at[par, :, pl.ds(0, 256)], sem_in
                ).wait()

            @pl.when((lo + i) != _NCH - 1)
            def _():
                pltpu.make_async_copy(
                    tab_t_hbm.at[:, pl.ds(0, _CW)], tblk.at[par], sem_in
                ).wait()
            cblk = lo + i
            par_v = jnp.full((_L,), par, jnp.int32)

            def match_body(v, u_tot):
                cvec = my_c[pl.ds(v * _L, _L)]
                valid = (lane + v * _L) < m_total
                m0 = jnp.logical_and(cvec == cblk, valid)
                s0 = jnp.sum(jnp.where(m0, one_v, zero_v))

                def emit_one(_, state):
                    m, u = state
                    f_s = jnp.min(jnp.where(m, lane, sent_v))
                    f = jnp.full((_L,), f_s, jnp.int32)
                    p = f + v * _L
                    lspl = plsc.load_gather(my_l, [p])
                    bspl = plsc.load_gather(my_b, [p])
                    slot = jax.lax.rem(u, jnp.int32(_NRB))

                    @pl.when(u >= _NRB)
                    def _():
                        pltpu.make_async_copy(
                            out_hbm.at[pl.ds(0, EMBED_DIM)],
                            rowring.at[pl.ds(0, EMBED_DIM)],
                            sem_out,
                        ).wait()

                    roff = pl.multiple_of(slot * EMBED_DIM, 8)
                    for g in range(EMBED_DIM // _L):
                        dvec = lane + g * _L
                        col = plsc.load_gather(tblk, [par_v, dvec, lspl])
                        rowring[pl.ds(roff + g * _L, _L)] = col
                    b0 = bspl[0]
                    boff = pl.multiple_of(b0 * EMBED_DIM, 8)
                    pltpu.async_copy(
                        rowring.at[pl.ds(roff, EMBED_DIM)],
                        out_hbm.at[pl.ds(boff, EMBED_DIM)],
                        sem_out,
                    )
                    m = jnp.logical_and(m, lane != f)
                    return m, u + 1

                _, u_tot = lax.fori_loop(0, s0, emit_one, (m0, u_tot),
                                         unroll=False)
                return u_tot

            return lax.fori_loop(0, nvec, match_body, u_tot, unroll=False)

        u_tot = lax.fori_loop(0, nch, col_body, jnp.int32(0), unroll=False)

        def drain_body(_, __):
            pltpu.make_async_copy(
                out_hbm.at[pl.ds(0, EMBED_DIM)],
                rowring.at[pl.ds(0, EMBED_DIM)],
                sem_out,
            ).wait()
            return ()

        lax.fori_loop(0, jnp.minimum(u_tot, _NRB), drain_body, (),
                      unroll=False)

    return gather_kernel


_gather = _make_gather()


def kernel(user_idx, table):
    tail_pad = jnp.pad(table[(_NCH - 1) * _CW + 128:].T,
                       ((0, 0), (0, 128 - _LASTW)))
    out1d = _gather(user_idx.astype(jnp.int32), table.T, tail_pad)
    return out1d.reshape(BATCH, EMBED_DIM)


# speedup vs baseline: 1.6994x; 1.0271x over previous
import functools

import jax
import jax.numpy as jnp
from jax import lax
from jax.experimental import pallas as pl
from jax.experimental.pallas import tpu as pltpu
from jax.experimental.pallas import tpu_sc as plsc

NUM_USERS = 100000
EMBED_DIM = 64
BATCH = 4096

_info = plsc.get_sparse_core_info()
_NC, _NS, _L = _info.num_cores, _info.num_subcores, _info.num_lanes
_NW = _NC * _NS
_CW = 512
_NCH = (NUM_USERS + _CW - 1) // _CW
_LASTW = NUM_USERS - (_NCH - 1) * _CW - 128
_CH_LO = _NCH // _NW
_CH_EXTRA = _NCH - _CH_LO * _NW
_NRB = 16
_NBUF = 3


def _make_gather():
    mesh = plsc.VectorSubcoreMesh(core_axis_name="c", subcore_axis_name="s")

    @functools.partial(
        pl.kernel,
        mesh=mesh,
        out_type=jax.ShapeDtypeStruct((BATCH * EMBED_DIM,), jnp.float32),
        scratch_types=[
            pltpu.VMEM((BATCH,), jnp.int32),
            pltpu.VMEM((BATCH,), jnp.int32),
            pltpu.VMEM((BATCH,), jnp.int32),
            pltpu.VMEM((BATCH,), jnp.int32),
            pltpu.VMEM((BATCH,), jnp.int32),
            pltpu.VMEM((_NBUF, EMBED_DIM, _CW), jnp.float32),
            pltpu.VMEM((_NRB * EMBED_DIM,), jnp.float32),
            pltpu.SemaphoreType.DMA,
            pltpu.SemaphoreType.DMA,
        ],
        compiler_params=pltpu.CompilerParams(needs_layout_passes=False),
    )
    def gather_kernel(idx_hbm, tab_t_hbm, tail_hbm, out_hbm, idx_v, my_b,
                      my_c, my_l, blk_v, tblk, rowring, sem_in, sem_out):
        wid = lax.axis_index("s") * _NC + lax.axis_index("c")
        nch = jnp.where(wid < _CH_EXTRA, _CH_LO + 1, _CH_LO)
        lo = wid * _CH_LO + jnp.minimum(wid, _CH_EXTRA)

        def start_blk(i):
            par = jax.lax.rem(i, jnp.int32(_NBUF))
            cblk = lo + i
            coff = pl.multiple_of(cblk * _CW, 128)

            @pl.when(cblk == _NCH - 1)
            def _():
                pltpu.async_copy(tab_t_hbm.at[:, pl.ds(coff, 128)],
                                 tblk.at[par, :, pl.ds(0, 128)], sem_in)
                pltpu.async_copy(tail_hbm,
                                 tblk.at[par, :, pl.ds(128, 128)], sem_in)

            @pl.when(cblk != _NCH - 1)
            def _():
                pltpu.async_copy(tab_t_hbm.at[:, pl.ds(coff, _CW)],
                                 tblk.at[par], sem_in)

        def prime_body(i, _):
            @pl.when(i < nch)
            def _():
                start_blk(i)
            return ()

        lax.fori_loop(0, _NBUF - 1, prime_body, (), unroll=True)

        pltpu.sync_copy(idx_hbm, idx_v)

        lane = lax.iota(jnp.int32, _L)
        one_v = jnp.full((_L,), 1, jnp.int32)
        zero_v = jnp.full((_L,), 0, jnp.int32)
        sent_v = jnp.full((_L,), _L, jnp.int32)

        def scan_body(v, off):
            vec = idx_v[pl.ds(v * _L, _L)]
            c = jax.lax.shift_right_logical(vec, 9)
            low = jnp.bitwise_and(vec, _CW - 1)
            b = lane + v * _L
            m = jnp.logical_and(c >= lo, c < lo + nch)
            m01 = jnp.where(m, one_v, zero_v)
            incl = plsc.cumsum(m01)
            pos = jnp.full((_L,), off, jnp.int32) + incl - m01
            plsc.store_scatter(my_b, [pos], b, mask=m)
            plsc.store_scatter(my_c, [pos], c, mask=m)
            plsc.store_scatter(my_l, [pos], low, mask=m)
            return off + jnp.sum(m01)

        m_total = lax.fori_loop(0, BATCH // _L, scan_body, jnp.int32(0),
                                unroll=4)
        nvec = (m_total + _L - 1) // _L

        def col_body(i, u_tot):
            par = jax.lax.rem(i, jnp.int32(_NBUF))

            @pl.when(i + _NBUF - 1 < nch)
            def _():
                start_blk(i + _NBUF - 1)

            @pl.when((lo + i) == _NCH - 1)
            def _():
                pltpu.make_async_copy(
                    tab_t_hbm.at[:, pl.ds(0, 256)],
                    tblk.at[par, :, pl.ds(0, 256)], sem_in
                ).wait()

            @pl.when((lo + i) != _NCH - 1)
            def _():
                pltpu.make_async_copy(
                    tab_t_hbm.at[:, pl.ds(0, _CW)], tblk.at[par], sem_in
                ).wait()
            cblk = lo + i
            par_v = jnp.full((_L,), par, jnp.int32)

            def match_body(v, off2):
                cvec = my_c[pl.ds(v * _L, _L)]
                bvec = my_b[pl.ds(v * _L, _L)]
                lvec = my_l[pl.ds(v * _L, _L)]
                valid = (lane + v * _L) < m_total
                m = jnp.logical_and(cvec == cblk, valid)
                m01 = jnp.where(m, one_v, zero_v)
                incl = plsc.cumsum(m01)
                pos = jnp.full((_L,), off2, jnp.int32) + incl - m01
                packed = jnp.bitwise_or(jnp.left_shift(bvec, 9), lvec)
                plsc.store_scatter(blk_v, [pos], packed, mask=m)
                return off2 + jnp.sum(m01)

            cnt = lax.fori_loop(0, nvec, match_body, jnp.int32(0),
                                unroll=False)

            def emit_one(j, state):
                u, drained = state
                vspl = plsc.load_gather(blk_v, [jnp.full((_L,), j, jnp.int32)])
                lspl = jnp.bitwise_and(vspl, _CW - 1)
                b0 = jnp.right_shift(vspl[0], 9)
                slot = jax.lax.rem(u, jnp.int32(_NRB))

                @pl.when(jnp.logical_and(slot == 0, u >= _NRB))
                def _():
                    pltpu.make_async_copy(
                        out_hbm.at[pl.ds(0, _NRB * EMBED_DIM)],
                        rowring,
                        sem_out,
                    ).wait()

                roff = pl.multiple_of(slot * EMBED_DIM, 8)
                for g in range(EMBED_DIM // _L):
                    dvec = lane + g * _L
                    col = plsc.load_gather(tblk, [par_v, dvec, lspl])
                    rowring[pl.ds(roff + g * _L, _L)] = col
                boff = pl.multiple_of(b0 * EMBED_DIM, 8)
                pltpu.async_copy(
                    rowring.at[pl.ds(roff, EMBED_DIM)],
                    out_hbm.at[pl.ds(boff, EMBED_DIM)],
                    sem_out,
                )
                drained = jnp.where(
                    jnp.logical_and(slot == 0, u >= _NRB),
                    drained + _NRB, drained)
                return u + 1, drained

            return lax.fori_loop(0, cnt, emit_one, u_tot, unroll=False)

        u_tot, drained = lax.fori_loop(
            0, nch, col_body, (jnp.int32(0), jnp.int32(0)), unroll=False)

        def drain_body(_, __):
            pltpu.make_async_copy(
                out_hbm.at[pl.ds(0, EMBED_DIM)],
                rowring.at[pl.ds(0, EMBED_DIM)],
                sem_out,
            ).wait()
            return ()

        lax.fori_loop(0, u_tot - drained, drain_body, (), unroll=False)

    return gather_kernel


_gather = _make_gather()


def kernel(user_idx, table):
    tail_pad = jnp.pad(table[(_NCH - 1) * _CW + 128:].T,
                       ((0, 0), (0, 128 - _LASTW)))
    out1d = _gather(user_idx.astype(jnp.int32), table.T, tail_pad)
    return out1d.reshape(BATCH, EMBED_DIM)
